# Initial kernel scaffold; baseline (speedup 1.0000x reference)
import functools

import jax
import jax.numpy as jnp
from jax import lax
from jax.experimental import pallas as pl
from jax.experimental.pallas import tpu as pltpu
from jax.experimental.pallas import tpu_sc as plsc

G = 4
K = 1024
CD = 64

N_TOKENS = 16 * 576
BN = 1152


def _tc_body(x_ref, cbt_ref, idx_ref, loss_ref):
    i = pl.program_id(0)

    @pl.when(i == 0)
    def _init():
        loss_ref[0, 0] = 0.0

    xb = x_ref[...]
    total = jnp.float32(0.0)
    for g in range(G):
        xg = xb[:, g * CD:(g + 1) * CD]
        cbt = cbt_ref[g]
        cross = jax.lax.dot_general(
            xg, cbt, (((1,), (0,)), ((), ())),
            preferred_element_type=jnp.float32)
        xsq = jnp.sum(xg * xg, axis=1, keepdims=True)
        csq = jnp.sum(cbt * cbt, axis=0, keepdims=True)
        d2 = jnp.maximum(xsq + csq - 2.0 * cross, 0.0)
        mn = jnp.min(d2, axis=1)
        iota = lax.broadcasted_iota(jnp.int32, (BN, K), 1)
        masked = jnp.where(d2 == mn[:, None], iota, K)
        idx_ref[g, :] = jnp.min(masked, axis=1)
        total = total + jnp.sum(mn)
    loss_ref[0, 0] += total


_SC_CHUNK = 128
_INFO = plsc.get_sparse_core_info()
_NW = _INFO.num_cores * _INFO.num_subcores
_B_PER_W = (N_TOKENS * G) // _NW
_NCH = _B_PER_W // _SC_CHUNK


def _sc_gather(cb_hbm, idx_hbm, out_hbm, idx_v, rows_v, sem):
    wid = lax.axis_index("s") * _INFO.num_cores + lax.axis_index("c")
    pltpu.sync_copy(idx_hbm.at[wid], idx_v)
    copies = [
        pltpu.async_copy(cb_hbm.at[idx_v.at[j]], rows_v.at[j], sem)
        for j in range(_NCH)
    ]
    for c in copies:
        c.wait()
    pltpu.sync_copy(rows_v, out_hbm.at[wid])


def kernel(x, codebook):
    B, T, D = x.shape
    x2d = x.reshape(N_TOKENS, D)
    cbt = codebook.transpose(0, 2, 1)

    num_blocks = N_TOKENS // BN
    idx_gn, loss_sum = pl.pallas_call(
        _tc_body,
        grid=(num_blocks,),
        in_specs=[
            pl.BlockSpec((BN, D), lambda i: (i, 0)),
            pl.BlockSpec((G, CD, K), lambda i: (0, 0, 0)),
        ],
        out_specs=[
            pl.BlockSpec((G, BN), lambda i: (0, i)),
            pl.BlockSpec((1, 1), lambda i: (0, 0)),
        ],
        out_shape=[
            jax.ShapeDtypeStruct((G, N_TOKENS), jnp.int32),
            jax.ShapeDtypeStruct((1, 1), jnp.float32),
        ],
        compiler_params=pltpu.CompilerParams(
            dimension_semantics=("arbitrary",),
        ),
    )(x2d, cbt)

    idx_ng = idx_gn.T
    flat_idx = (idx_ng + jnp.arange(G, dtype=jnp.int32)[None, :] * K)
    flat_idx = flat_idx.reshape(_NW, _NCH, _SC_CHUNK)

    mesh = plsc.VectorSubcoreMesh(core_axis_name="c", subcore_axis_name="s")
    gathered = pl.kernel(
        _sc_gather,
        mesh=mesh,
        out_type=jax.ShapeDtypeStruct((_NW, _NCH, _SC_CHUNK, CD), jnp.float32),
        scratch_types=[
            pltpu.VMEM((_NCH, _SC_CHUNK), jnp.int32),
            pltpu.VMEM((_NCH, _SC_CHUNK, CD), jnp.float32),
            pltpu.SemaphoreType.DMA,
        ],
    )(codebook.reshape(G * K, CD), flat_idx)

    quantized = gathered.reshape(B, T, D)
    loss = loss_sum[0, 0] / jnp.float32(N_TOKENS * G * CD)
    indices = idx_ng.reshape(B, T, G)
    return quantized, loss, loss, indices


# speedup vs baseline: 6.0997x; 6.0997x over previous
import functools

import jax
import jax.numpy as jnp
from jax import lax
from jax.experimental import pallas as pl
from jax.experimental.pallas import tpu as pltpu
from jax.experimental.pallas import tpu_sc as plsc

G = 4
K = 1024
CD = 64

N_TOKENS = 16 * 576
BN = 1152


def _tc_body(x_ref, cbt_ref, idx_ref, loss_ref):
    i = pl.program_id(0)

    @pl.when(i == 0)
    def _init():
        loss_ref[...] = jnp.zeros_like(loss_ref)

    xb = x_ref[...]
    total = jnp.float32(0.0)
    for g in range(G):
        xg = xb[:, g * CD:(g + 1) * CD]
        cbt = cbt_ref[g]
        cross = jax.lax.dot_general(
            xg, cbt, (((1,), (0,)), ((), ())),
            preferred_element_type=jnp.float32)
        xsq = jnp.sum(xg * xg, axis=1, keepdims=True)
        csq = jnp.sum(cbt * cbt, axis=0, keepdims=True)
        d2 = jnp.maximum(xsq + csq - 2.0 * cross, 0.0)
        mn = jnp.min(d2, axis=1)
        iota = lax.broadcasted_iota(jnp.int32, (BN, K), 1)
        masked = jnp.where(d2 == mn[:, None], iota, K)
        idx_ref[g, :] = jnp.min(masked, axis=1)
        total = total + jnp.sum(mn)
    loss_ref[...] += jnp.reshape(total, (1, 1))


_SC_CHUNK = 128
_NUM_SC = 2
_NUM_SUBCORES = 16
_NW = _NUM_SC * _NUM_SUBCORES
_B_PER_W = (N_TOKENS * G) // _NW
_NCH = _B_PER_W // _SC_CHUNK


def _sc_gather(cb_hbm, idx_hbm, out_hbm, idx_v, rows_v, sem):
    wid = lax.axis_index("s") * _NUM_SC + lax.axis_index("c")
    pltpu.sync_copy(idx_hbm.at[wid], idx_v)
    copies = [
        pltpu.async_copy(cb_hbm.at[idx_v.at[j]], rows_v.at[j], sem)
        for j in range(_NCH)
    ]
    for c in copies:
        c.wait()
    pltpu.sync_copy(rows_v, out_hbm.at[wid])


def kernel(x, codebook):
    B, T, D = x.shape
    x2d = x.reshape(N_TOKENS, D)
    cbt = codebook.transpose(0, 2, 1)

    num_blocks = N_TOKENS // BN
    idx_gn, loss_sum = pl.pallas_call(
        _tc_body,
        grid=(num_blocks,),
        in_specs=[
            pl.BlockSpec((BN, D), lambda i: (i, 0)),
            pl.BlockSpec((G, CD, K), lambda i: (0, 0, 0)),
        ],
        out_specs=[
            pl.BlockSpec((G, BN), lambda i: (0, i)),
            pl.BlockSpec((1, 1), lambda i: (0, 0)),
        ],
        out_shape=[
            jax.ShapeDtypeStruct((G, N_TOKENS), jnp.int32),
            jax.ShapeDtypeStruct((1, 1), jnp.float32),
        ],
        compiler_params=pltpu.CompilerParams(
            dimension_semantics=("arbitrary",),
        ),
    )(x2d, cbt)

    idx_ng = idx_gn.T
    flat_idx = (idx_ng + jnp.arange(G, dtype=jnp.int32)[None, :] * K)
    flat_idx = flat_idx.reshape(_NW, _NCH, _SC_CHUNK)

    mesh = plsc.VectorSubcoreMesh(core_axis_name="c", subcore_axis_name="s")
    gathered = pl.kernel(
        _sc_gather,
        mesh=mesh,
        out_type=jax.ShapeDtypeStruct((_NW, _NCH, _SC_CHUNK, CD), jnp.float32),
        scratch_types=[
            pltpu.VMEM((_NCH, _SC_CHUNK), jnp.int32),
            pltpu.VMEM((_NCH, _SC_CHUNK, CD), jnp.float32),
            pltpu.SemaphoreType.DMA,
        ],
        compiler_params=pltpu.CompilerParams(use_tc_tiling_on_sc=False),
    )(codebook.reshape(G * K, CD), flat_idx)

    quantized = gathered.reshape(B, T, D)
    loss = loss_sum[0, 0] / jnp.float32(N_TOKENS * G * CD)
    indices = idx_ng.reshape(B, T, G)
    return quantized, loss, loss, indices


# speedup vs baseline: 7.3967x; 1.2126x over previous
import functools

import jax
import jax.numpy as jnp
from jax import lax
from jax.experimental import pallas as pl
from jax.experimental.pallas import tpu as pltpu
from jax.experimental.pallas import tpu_sc as plsc

G = 4
K = 1024
CD = 64

N_TOKENS = 16 * 576
BN = 1152


def _tc_body(x_ref, cbt_ref, idx_ref, fid_ref, loss_ref):
    i = pl.program_id(0)

    @pl.when(i == 0)
    def _init():
        loss_ref[...] = jnp.zeros_like(loss_ref)

    xb = x_ref[...]
    iota1 = lax.broadcasted_iota(jnp.int32, (1, K), 1).astype(jnp.float32)
    total = jnp.float32(0.0)
    for g in range(G):
        xg = xb[:, g * CD:(g + 1) * CD]
        cbt = cbt_ref[g]
        cross = jax.lax.dot_general(
            xg, cbt, (((1,), (0,)), ((), ())),
            preferred_element_type=jnp.float32)
        xsq = jnp.sum(xg * xg, axis=1, keepdims=True)
        csq = jnp.sum(cbt * cbt, axis=0, keepdims=True)
        d2 = xsq + csq - 2.0 * cross
        mn = jnp.min(d2, axis=1, keepdims=True)
        masked = jnp.where(d2 == mn, iota1, jnp.float32(K))
        idx_f = jnp.min(masked, axis=1, keepdims=True)
        idx = idx_f.astype(jnp.int32)
        idx_ref[:, g:g + 1] = idx
        fid_ref[:, g:g + 1] = idx + g * K
        total = total + jnp.sum(jnp.maximum(mn, 0.0))
    loss_ref[...] += jnp.reshape(total, (1, 1))


_SC_CHUNK = 128
_NUM_SC = 2
_NUM_SUBCORES = 16
_NW = _NUM_SC * _NUM_SUBCORES
_B_PER_W = (N_TOKENS * G) // _NW
_NCH = _B_PER_W // _SC_CHUNK


def _sc_gather(cb_hbm, idx_hbm, out_hbm, idx_v, rows_v, sem):
    wid = lax.axis_index("s") * _NUM_SC + lax.axis_index("c")
    pltpu.sync_copy(idx_hbm.at[wid], idx_v)
    copies = [
        pltpu.async_copy(cb_hbm.at[idx_v.at[j]], rows_v.at[j], sem)
        for j in range(_NCH)
    ]
    for c in copies:
        c.wait()
    pltpu.sync_copy(rows_v, out_hbm.at[wid])


def kernel(x, codebook):
    B, T, D = x.shape
    x2d = x.reshape(N_TOKENS, D)
    cbt = codebook.transpose(0, 2, 1)

    num_blocks = N_TOKENS // BN
    idx_ng, fid_ng, loss_sum = pl.pallas_call(
        _tc_body,
        grid=(num_blocks,),
        in_specs=[
            pl.BlockSpec((BN, D), lambda i: (i, 0)),
            pl.BlockSpec((G, CD, K), lambda i: (0, 0, 0)),
        ],
        out_specs=[
            pl.BlockSpec((BN, G), lambda i: (i, 0)),
            pl.BlockSpec((BN, G), lambda i: (i, 0)),
            pl.BlockSpec((1, 1), lambda i: (0, 0)),
        ],
        out_shape=[
            jax.ShapeDtypeStruct((N_TOKENS, G), jnp.int32),
            jax.ShapeDtypeStruct((N_TOKENS, G), jnp.int32),
            jax.ShapeDtypeStruct((1, 1), jnp.float32),
        ],
        compiler_params=pltpu.CompilerParams(
            dimension_semantics=("arbitrary",),
        ),
    )(x2d, cbt)

    flat_idx = fid_ng.reshape(_NW, _NCH, _SC_CHUNK)

    mesh = plsc.VectorSubcoreMesh(core_axis_name="c", subcore_axis_name="s")
    gathered = pl.kernel(
        _sc_gather,
        mesh=mesh,
        out_type=jax.ShapeDtypeStruct((_NW, _NCH, _SC_CHUNK, CD), jnp.float32),
        scratch_types=[
            pltpu.VMEM((_NCH, _SC_CHUNK), jnp.int32),
            pltpu.VMEM((_NCH, _SC_CHUNK, CD), jnp.float32),
            pltpu.SemaphoreType.DMA,
        ],
        compiler_params=pltpu.CompilerParams(use_tc_tiling_on_sc=False),
    )(codebook.reshape(G * K, CD), flat_idx)

    quantized = gathered.reshape(B, T, D)
    loss = loss_sum[0, 0] / jnp.float32(N_TOKENS * G * CD)
    indices = idx_ng.reshape(B, T, G)
    return quantized, loss, loss, indices
